# baseline (speedup 1.0000x reference)
import functools

import jax
import jax.numpy as jnp
from jax import lax
from jax.experimental import pallas as pl
from jax.experimental.pallas import tpu as pltpu
from jax.experimental.pallas import tpu_sc as plsc

B = 16384
D = 32
V = 1000000
CTX = 20
NEG = 20
R = CTX + NEG
NC = 2
NS = 16
NW = NC * NS
BPW = B // NW
CB = 32
NCHUNK = BPW // CB

VB = 768
BR = VB // 4
NBLK = V // VB
VTAIL = V - NBLK * VB
VP = V // 4
BLK_PER_W = -(-NBLK // NW)


def _transpose_one(src_hbm, dst_hbm, wid, in0_v, in1_v, out0_v, out1_v,
                   scat_v, sem_i0, sem_i1, sem_o0, sem_o1):
    in_v = (in0_v, in1_v)
    out_v = (out0_v, out1_v)
    sem_i = (sem_i0, sem_i1)
    sem_o = (sem_o0, sem_o1)

    NPAIR = (BLK_PER_W + 2) // 2

    pltpu.async_copy(src_hbm.at[:, pl.ds(wid * VB, VB)], in_v[0], sem_i[0])
    pltpu.async_copy(src_hbm.at[:, pl.ds((wid + NW) * VB, VB)], in_v[1],
                     sem_i[1])

    def pair_body(g, carry):
        for buf in (0, 1):
            blk = wid + (2 * g + buf) * NW

            @pl.when(blk < NBLK)
            def _(blk=blk, buf=buf):
                pltpu.make_async_copy(
                    src_hbm.at[:, pl.ds(0, VB)], in_v[buf],
                    sem_i[buf]).wait()

                @pl.when(g >= 1)
                def _():
                    pltpu.make_async_copy(
                        out_v[buf], dst_hbm.at[pl.ds(0, BR)],
                        sem_o[buf]).wait()

                lanes33 = [jax.lax.iota(jnp.int32, 16) * 33 + d
                           for d in range(D)]

                def scat_body(k, carry2):
                    base = scat_v.at[pl.ds(k * 528, 528)]
                    for d in range(D):
                        plsc.store_scatter(
                            base, [lanes33[d]],
                            in_v[buf][d, pl.ds(k * 16, 16)])
                    return carry2

                lax.fori_loop(0, VB // 16, scat_body, 0, unroll=2)

                def compact_body(k, carry2):
                    for dv in range(16):
                        vloc = k * 16 + dv
                        for dh in range(2):
                            out_v[buf][4 * k + dv // 4,
                                       pl.ds(32 * (dv % 4) + dh * 16, 16)] = (
                                scat_v[pl.ds(vloc * 33 + dh * 16, 16)])
                    return carry2

                lax.fori_loop(0, VB // 16, compact_body, 0, unroll=2)
                pltpu.async_copy(
                    out_v[buf], dst_hbm.at[pl.ds(blk * BR, BR)], sem_o[buf])

            @pl.when(blk + 2 * NW < NBLK)
            def _(blk=blk, buf=buf):
                pltpu.async_copy(
                    src_hbm.at[:, pl.ds((blk + 2 * NW) * VB, VB)],
                    in_v[buf], sem_i[buf])
        return carry

    lax.fori_loop(0, NPAIR, pair_body, 0)

    for i in range(2 * NPAIR - 3, 2 * NPAIR):
        buf = i % 2
        blk = wid + i * NW

        @pl.when(jnp.logical_and(blk < NBLK, blk + 2 * NW >= NBLK))
        def _(buf=buf):
            pltpu.make_async_copy(
                out_v[buf], dst_hbm.at[pl.ds(0, BR)], sem_o[buf]).wait()


def _sc_transpose_body(wcen_t_hbm, wctx_t_hbm, tail_cen_hbm, tail_ctx_hbm,
                       cen_out_hbm, ctx_out_hbm,
                       in0_v, in1_v, out0_v, out1_v, scat_v, tail_v,
                       sem_i0, sem_i1, sem_o0, sem_o1):
    wid = lax.axis_index("s") * NC + lax.axis_index("c")
    _transpose_one(wcen_t_hbm, cen_out_hbm, wid, in0_v, in1_v, out0_v,
                   out1_v, scat_v, sem_i0, sem_i1, sem_o0, sem_o1)
    _transpose_one(wctx_t_hbm, ctx_out_hbm, wid, in0_v, in1_v, out0_v,
                   out1_v, scat_v, sem_i0, sem_i1, sem_o0, sem_o1)

    @pl.when(wid == NW - 1)
    def _():
        for tail_hbm, dst_hbm in ((tail_cen_hbm, cen_out_hbm),
                                  (tail_ctx_hbm, ctx_out_hbm)):
            pltpu.sync_copy(tail_hbm, tail_v)
            pltpu.sync_copy(tail_v, dst_hbm.at[pl.ds(NBLK * BR, VTAIL // 4)])


def _sc_expand_body(cen_p_hbm, ctx_p_hbm, cen_out_hbm, ctx_out_hbm,
                    pin0_v, pin1_v, row0_v, row1_v,
                    sem_i0, sem_i1, sem_o0, sem_o1):
    wid = lax.axis_index("s") * NC + lax.axis_index("c")
    pin_v = (pin0_v, pin1_v)
    row_v = (row0_v, row1_v)
    sem_i = (sem_i0, sem_i1)
    sem_o = (sem_o0, sem_o1)

    NPAIR = (BLK_PER_W + 2) // 2

    for src_hbm, dst_hbm in ((cen_p_hbm, cen_out_hbm),
                             (ctx_p_hbm, ctx_out_hbm)):
        pltpu.async_copy(src_hbm.at[pl.ds(wid * BR, BR)], pin_v[0], sem_i[0])
        pltpu.async_copy(src_hbm.at[pl.ds((wid + NW) * BR, BR)], pin_v[1],
                         sem_i[1])

        def pair_body(g, carry, src_hbm=src_hbm, dst_hbm=dst_hbm):
            for buf in (0, 1):
                blk = wid + (2 * g + buf) * NW

                @pl.when(blk < NBLK)
                def _(blk=blk, buf=buf):
                    pltpu.make_async_copy(
                        src_hbm.at[pl.ds(0, BR)], pin_v[buf],
                        sem_i[buf]).wait()

                    @pl.when(g >= 1)
                    def _():
                        pltpu.make_async_copy(
                            row_v[buf], dst_hbm.at[pl.ds(0, VB)],
                            sem_o[buf]).wait()

                    def mv(k, carry2):
                        for t in range(16):
                            w = k * 16 + t
                            row_v[buf][w // 2, pl.ds((w % 2) * 16, 16)] = (
                                pin_v[buf][w // 8, pl.ds((w % 8) * 16, 16)])
                        return carry2

                    lax.fori_loop(0, BR * 8 // 16, mv, 0)
                    pltpu.async_copy(
                        row_v[buf], dst_hbm.at[pl.ds(blk * VB, VB)],
                        sem_o[buf])

                @pl.when(blk + 2 * NW < NBLK)
                def _(blk=blk, buf=buf):
                    pltpu.async_copy(
                        src_hbm.at[pl.ds((blk + 2 * NW) * BR, BR)],
                        pin_v[buf], sem_i[buf])
            return carry

        lax.fori_loop(0, NPAIR, pair_body, 0)

        for i in range(2 * NPAIR - 3, 2 * NPAIR):
            buf = i % 2
            blk = wid + i * NW

            @pl.when(jnp.logical_and(blk < NBLK, blk + 2 * NW >= NBLK))
            def _(buf=buf, dst_hbm=dst_hbm):
                pltpu.make_async_copy(
                    row_v[buf], dst_hbm.at[pl.ds(0, VB)], sem_o[buf]).wait()

    @pl.when(wid == NW - 1)
    def _():
        for src_hbm, dst_hbm in ((cen_p_hbm, cen_out_hbm),
                                 (ctx_p_hbm, ctx_out_hbm)):
            pltpu.sync_copy(src_hbm.at[pl.ds(NBLK * BR, VTAIL // 4)],
                            pin0_v.at[pl.ds(0, VTAIL // 4)])

            def mv(w, carry):
                row0_v[w // 2, pl.ds((w % 2) * 16, 16)] = (
                    pin0_v[w // 8, pl.ds((w % 8) * 16, 16)])
                return carry

            lax.fori_loop(0, VTAIL * 2, mv, 0)
            pltpu.sync_copy(row0_v.at[pl.ds(0, VTAIL)],
                            dst_hbm.at[pl.ds(NBLK * VB, VTAIL)])


def _sc_scores_body(center_hbm, ctxneg_hbm, wcen_hbm, wctx_hbm,
                    pos_hbm, neg_hbm,
                    cidx_v, ridx_v, crow0_v, crow1_v, rrow0_v, rrow1_v,
                    pos_v, neg_v,
                    sem_c0, sem_c1, sem_r0, sem_r1):
    wid = lax.axis_index("s") * NC + lax.axis_index("c")
    base = wid * BPW

    pltpu.sync_copy(center_hbm.at[pl.ds(base, BPW)], cidx_v)
    pltpu.sync_copy(ctxneg_hbm.at[pl.ds(base * R, BPW * R)], ridx_v)

    crow = (crow0_v, crow1_v)
    rrow = (rrow0_v, rrow1_v)
    sem_c = (sem_c0, sem_c1)
    sem_r = (sem_r0, sem_r1)

    def start_gather(g):
        buf = g % 2
        cc = pltpu.async_copy(
            wcen_hbm.at[cidx_v.at[pl.ds(g * CB, CB)]], crow[buf], sem_c[buf])
        cr = pltpu.async_copy(
            wctx_hbm.at[ridx_v.at[pl.ds(g * CB * R, CB * R)]], rrow[buf],
            sem_r[buf])
        return cc, cr

    pending = {0: start_gather(0)}

    for g in range(NCHUNK):
        buf = g % 2
        if g + 1 < NCHUNK:
            pending[g + 1] = start_gather(g + 1)
        cc, cr = pending.pop(g)
        cc.wait()
        cr.wait()
        rrow_v = rrow[buf]
        crow_v = crow[buf]

        def elem_body(b, carry2, rrow_v=rrow_v, crow_v=crow_v, g=g):
            rb = b * R
            accp0 = rrow_v[rb, pl.ds(0, 16)]
            accp1 = rrow_v[rb, pl.ds(16, 16)]
            for j in range(1, CTX):
                accp0 = accp0 + rrow_v[rb + j, pl.ds(0, 16)]
                accp1 = accp1 + rrow_v[rb + j, pl.ds(16, 16)]
            accn0 = rrow_v[rb + CTX, pl.ds(0, 16)]
            accn1 = rrow_v[rb + CTX, pl.ds(16, 16)]
            for j in range(CTX + 1, R):
                accn0 = accn0 + rrow_v[rb + j, pl.ds(0, 16)]
                accn1 = accn1 + rrow_v[rb + j, pl.ds(16, 16)]
            c0 = crow_v[b, pl.ds(0, 16)]
            c1 = crow_v[b, pl.ds(16, 16)]
            pos_v[g * CB + b, pl.ds(0, 16)] = accp0 * c0 + accp1 * c1
            neg_v[g * CB + b, pl.ds(0, 16)] = accn0 * c0 + accn1 * c1
            return carry2

        lax.fori_loop(0, CB, elem_body, 0)

    pltpu.sync_copy(pos_v, pos_hbm.at[pl.ds(base, BPW)])
    pltpu.sync_copy(neg_v, neg_hbm.at[pl.ds(base, BPW)])


def _tc_loss_body(pos_ref, neg_ref, out_ref):
    p = jnp.sum(pos_ref[...], axis=1)
    n = jnp.sum(neg_ref[...], axis=1)
    ls = jax.nn.log_sigmoid(p) + jax.nn.log_sigmoid(-n)
    out_ref[...] = (-jnp.sum(ls) / B).reshape(1, 1)


@jax.jit
def kernel(center, context, negative, W_center, W_context):
    center = center.astype(jnp.int32)
    ctxneg = jnp.concatenate(
        [context.astype(jnp.int32), negative.astype(jnp.int32)], axis=1
    ).reshape(-1)

    mesh = plsc.VectorSubcoreMesh(core_axis_name="c", subcore_axis_name="s")

    sc_transpose = pl.kernel(
        _sc_transpose_body,
        out_type=[
            jax.ShapeDtypeStruct((VP, 128), jnp.float32),
            jax.ShapeDtypeStruct((VP, 128), jnp.float32),
        ],
        mesh=mesh,
        compiler_params=pltpu.CompilerParams(
            use_tc_tiling_on_sc=True, needs_layout_passes=False),
        scratch_types=[
            pltpu.VMEM((D, VB), jnp.float32),
            pltpu.VMEM((D, VB), jnp.float32),
            pltpu.VMEM((BR, 128), jnp.float32),
            pltpu.VMEM((BR, 128), jnp.float32),
            pltpu.VMEM((VB * 33,), jnp.float32),
            pltpu.VMEM((VTAIL // 4, 128), jnp.float32),
            pltpu.SemaphoreType.DMA,
            pltpu.SemaphoreType.DMA,
            pltpu.SemaphoreType.DMA,
            pltpu.SemaphoreType.DMA,
        ],
    )
    cen_packed, ctx_packed = sc_transpose(
        W_center.T, W_context.T,
        W_center[NBLK * VB:].reshape(VTAIL // 4, 128),
        W_context[NBLK * VB:].reshape(VTAIL // 4, 128))

    sc_expand = pl.kernel(
        _sc_expand_body,
        out_type=[
            jax.ShapeDtypeStruct((V, D), jnp.float32),
            jax.ShapeDtypeStruct((V, D), jnp.float32),
        ],
        mesh=mesh,
        compiler_params=pltpu.CompilerParams(use_tc_tiling_on_sc=False),
        scratch_types=[
            pltpu.VMEM((BR, 128), jnp.float32),
            pltpu.VMEM((BR, 128), jnp.float32),
            pltpu.VMEM((VB, D), jnp.float32),
            pltpu.VMEM((VB, D), jnp.float32),
            pltpu.SemaphoreType.DMA,
            pltpu.SemaphoreType.DMA,
            pltpu.SemaphoreType.DMA,
            pltpu.SemaphoreType.DMA,
        ],
    )
    W_center, W_context = sc_expand(cen_packed, ctx_packed)
    sc_scores = pl.kernel(
        _sc_scores_body,
        out_type=[
            jax.ShapeDtypeStruct((B, 16), jnp.float32),
            jax.ShapeDtypeStruct((B, 16), jnp.float32),
        ],
        mesh=mesh,
        compiler_params=pltpu.CompilerParams(use_tc_tiling_on_sc=False),
        scratch_types=[
            pltpu.VMEM((BPW,), jnp.int32),
            pltpu.VMEM((BPW * R,), jnp.int32),
            pltpu.VMEM((CB, D), jnp.float32),
            pltpu.VMEM((CB, D), jnp.float32),
            pltpu.VMEM((CB * R, D), jnp.float32),
            pltpu.VMEM((CB * R, D), jnp.float32),
            pltpu.VMEM((BPW, 16), jnp.float32),
            pltpu.VMEM((BPW, 16), jnp.float32),
            pltpu.SemaphoreType.DMA,
            pltpu.SemaphoreType.DMA,
            pltpu.SemaphoreType.DMA,
            pltpu.SemaphoreType.DMA,
        ],
    )
    pos, neg = sc_scores(center, ctxneg, W_center, W_context)

    loss2d = pl.pallas_call(
        _tc_loss_body,
        out_shape=jax.ShapeDtypeStruct((1, 1), jnp.float32),
    )(pos, neg)
    return loss2d[0, 0]


# speedup vs baseline: 2.5154x; 2.5154x over previous
import functools

import jax
import jax.numpy as jnp
from jax import lax
from jax.experimental import pallas as pl
from jax.experimental.pallas import tpu as pltpu
from jax.experimental.pallas import tpu_sc as plsc

B = 16384
D = 32
V = 1000000
CTX = 20
NEG = 20
R = CTX + NEG
NC = 2
NS = 16
NW = NC * NS
BPW = B // NW
CB = 32
NCHUNK = BPW // CB

VB = 768
BR = VB // 4
NBLK = V // VB
VTAIL = V - NBLK * VB
VP = V // 4
BLK_PER_W = -(-NBLK // NW)


def _transpose_one(src_hbm, dst_hbm, wid, in0_v, in1_v, out0_v, out1_v,
                   scat_v, sem_i0, sem_i1, sem_o0, sem_o1):
    in_v = (in0_v, in1_v)
    out_v = (out0_v, out1_v)
    sem_i = (sem_i0, sem_i1)
    sem_o = (sem_o0, sem_o1)

    NPAIR = (BLK_PER_W + 2) // 2

    pltpu.async_copy(src_hbm.at[:, pl.ds(wid * VB, VB)], in_v[0], sem_i[0])
    pltpu.async_copy(src_hbm.at[:, pl.ds((wid + NW) * VB, VB)], in_v[1],
                     sem_i[1])

    def pair_body(g, carry):
        for buf in (0, 1):
            blk = wid + (2 * g + buf) * NW

            @pl.when(blk < NBLK)
            def _(blk=blk, buf=buf):
                pltpu.make_async_copy(
                    src_hbm.at[:, pl.ds(0, VB)], in_v[buf],
                    sem_i[buf]).wait()

                @pl.when(g >= 1)
                def _():
                    pltpu.make_async_copy(
                        out_v[buf], dst_hbm.at[pl.ds(0, BR)],
                        sem_o[buf]).wait()

                lanes33 = [jax.lax.iota(jnp.int32, 16) * 33 + d
                           for d in range(D)]

                @functools.partial(plsc.parallel_loop, 0, VB // 16, unroll=2)
                def _(k):
                    base = scat_v.at[pl.ds(k * 528, 528)]
                    for d in range(D):
                        plsc.store_scatter(
                            base, [lanes33[d]],
                            in_v[buf][d, pl.ds(k * 16, 16)])

                @functools.partial(plsc.parallel_loop, 0, VB // 16, unroll=2)
                def _(k):
                    for dv in range(16):
                        vloc = k * 16 + dv
                        for dh in range(2):
                            out_v[buf][4 * k + dv // 4,
                                       pl.ds(32 * (dv % 4) + dh * 16, 16)] = (
                                scat_v[pl.ds(vloc * 33 + dh * 16, 16)])
                pltpu.async_copy(
                    out_v[buf], dst_hbm.at[pl.ds(blk * BR, BR)], sem_o[buf])

            @pl.when(blk + 2 * NW < NBLK)
            def _(blk=blk, buf=buf):
                pltpu.async_copy(
                    src_hbm.at[:, pl.ds((blk + 2 * NW) * VB, VB)],
                    in_v[buf], sem_i[buf])
        return carry

    lax.fori_loop(0, NPAIR, pair_body, 0)

    for i in range(2 * NPAIR - 3, 2 * NPAIR):
        buf = i % 2
        blk = wid + i * NW

        @pl.when(jnp.logical_and(blk < NBLK, blk + 2 * NW >= NBLK))
        def _(buf=buf):
            pltpu.make_async_copy(
                out_v[buf], dst_hbm.at[pl.ds(0, BR)], sem_o[buf]).wait()


def _sc_transpose_body(wcen_t_hbm, wctx_t_hbm, tail_cen_hbm, tail_ctx_hbm,
                       cen_out_hbm, ctx_out_hbm,
                       in0_v, in1_v, out0_v, out1_v, scat_v, tail_v,
                       sem_i0, sem_i1, sem_o0, sem_o1):
    wid = lax.axis_index("s") * NC + lax.axis_index("c")
    _transpose_one(wcen_t_hbm, cen_out_hbm, wid, in0_v, in1_v, out0_v,
                   out1_v, scat_v, sem_i0, sem_i1, sem_o0, sem_o1)
    _transpose_one(wctx_t_hbm, ctx_out_hbm, wid, in0_v, in1_v, out0_v,
                   out1_v, scat_v, sem_i0, sem_i1, sem_o0, sem_o1)

    @pl.when(wid == NW - 1)
    def _():
        for tail_hbm, dst_hbm in ((tail_cen_hbm, cen_out_hbm),
                                  (tail_ctx_hbm, ctx_out_hbm)):
            pltpu.sync_copy(tail_hbm, tail_v)
            pltpu.sync_copy(tail_v, dst_hbm.at[pl.ds(NBLK * BR, VTAIL // 4)])


def _sc_expand_body(cen_p_hbm, ctx_p_hbm, cen_out_hbm, ctx_out_hbm,
                    pin0_v, pin1_v, row0_v, row1_v,
                    sem_i0, sem_i1, sem_o0, sem_o1):
    wid = lax.axis_index("s") * NC + lax.axis_index("c")
    pin_v = (pin0_v, pin1_v)
    row_v = (row0_v, row1_v)
    sem_i = (sem_i0, sem_i1)
    sem_o = (sem_o0, sem_o1)

    NPAIR = (BLK_PER_W + 2) // 2

    for src_hbm, dst_hbm in ((cen_p_hbm, cen_out_hbm),
                             (ctx_p_hbm, ctx_out_hbm)):
        pltpu.async_copy(src_hbm.at[pl.ds(wid * BR, BR)], pin_v[0], sem_i[0])
        pltpu.async_copy(src_hbm.at[pl.ds((wid + NW) * BR, BR)], pin_v[1],
                         sem_i[1])

        def pair_body(g, carry, src_hbm=src_hbm, dst_hbm=dst_hbm):
            for buf in (0, 1):
                blk = wid + (2 * g + buf) * NW

                @pl.when(blk < NBLK)
                def _(blk=blk, buf=buf):
                    pltpu.make_async_copy(
                        src_hbm.at[pl.ds(0, BR)], pin_v[buf],
                        sem_i[buf]).wait()

                    @pl.when(g >= 1)
                    def _():
                        pltpu.make_async_copy(
                            row_v[buf], dst_hbm.at[pl.ds(0, VB)],
                            sem_o[buf]).wait()

                    def mv(k, carry2):
                        for t in range(16):
                            w = k * 16 + t
                            row_v[buf][w // 2, pl.ds((w % 2) * 16, 16)] = (
                                pin_v[buf][w // 8, pl.ds((w % 8) * 16, 16)])
                        return carry2

                    lax.fori_loop(0, BR * 8 // 16, mv, 0)
                    pltpu.async_copy(
                        row_v[buf], dst_hbm.at[pl.ds(blk * VB, VB)],
                        sem_o[buf])

                @pl.when(blk + 2 * NW < NBLK)
                def _(blk=blk, buf=buf):
                    pltpu.async_copy(
                        src_hbm.at[pl.ds((blk + 2 * NW) * BR, BR)],
                        pin_v[buf], sem_i[buf])
            return carry

        lax.fori_loop(0, NPAIR, pair_body, 0)

        for i in range(2 * NPAIR - 3, 2 * NPAIR):
            buf = i % 2
            blk = wid + i * NW

            @pl.when(jnp.logical_and(blk < NBLK, blk + 2 * NW >= NBLK))
            def _(buf=buf, dst_hbm=dst_hbm):
                pltpu.make_async_copy(
                    row_v[buf], dst_hbm.at[pl.ds(0, VB)], sem_o[buf]).wait()

    @pl.when(wid == NW - 1)
    def _():
        for src_hbm, dst_hbm in ((cen_p_hbm, cen_out_hbm),
                                 (ctx_p_hbm, ctx_out_hbm)):
            pltpu.sync_copy(src_hbm.at[pl.ds(NBLK * BR, VTAIL // 4)],
                            pin0_v.at[pl.ds(0, VTAIL // 4)])

            def mv(w, carry):
                row0_v[w // 2, pl.ds((w % 2) * 16, 16)] = (
                    pin0_v[w // 8, pl.ds((w % 8) * 16, 16)])
                return carry

            lax.fori_loop(0, VTAIL * 2, mv, 0)
            pltpu.sync_copy(row0_v.at[pl.ds(0, VTAIL)],
                            dst_hbm.at[pl.ds(NBLK * VB, VTAIL)])


def _sc_scores_body(center_hbm, ctxneg_hbm, wcen_hbm, wctx_hbm,
                    pos_hbm, neg_hbm,
                    cidx_v, ridx_v, crow0_v, crow1_v, rrow0_v, rrow1_v,
                    pos_v, neg_v,
                    sem_c0, sem_c1, sem_r0, sem_r1):
    wid = lax.axis_index("s") * NC + lax.axis_index("c")
    base = wid * BPW

    pltpu.sync_copy(center_hbm.at[pl.ds(base, BPW)], cidx_v)
    pltpu.sync_copy(ctxneg_hbm.at[pl.ds(base * R, BPW * R)], ridx_v)

    crow = (crow0_v, crow1_v)
    rrow = (rrow0_v, rrow1_v)
    sem_c = (sem_c0, sem_c1)
    sem_r = (sem_r0, sem_r1)

    def start_gather(g):
        buf = g % 2
        cc = pltpu.async_copy(
            wcen_hbm.at[cidx_v.at[pl.ds(g * CB, CB)]], crow[buf], sem_c[buf])
        cr = pltpu.async_copy(
            wctx_hbm.at[ridx_v.at[pl.ds(g * CB * R, CB * R)]], rrow[buf],
            sem_r[buf])
        return cc, cr

    pending = {0: start_gather(0)}

    for g in range(NCHUNK):
        buf = g % 2
        if g + 1 < NCHUNK:
            pending[g + 1] = start_gather(g + 1)
        cc, cr = pending.pop(g)
        cc.wait()
        cr.wait()
        rrow_v = rrow[buf]
        crow_v = crow[buf]

        def elem_body(b, carry2, rrow_v=rrow_v, crow_v=crow_v, g=g):
            rb = b * R
            accp0 = rrow_v[rb, pl.ds(0, 16)]
            accp1 = rrow_v[rb, pl.ds(16, 16)]
            for j in range(1, CTX):
                accp0 = accp0 + rrow_v[rb + j, pl.ds(0, 16)]
                accp1 = accp1 + rrow_v[rb + j, pl.ds(16, 16)]
            accn0 = rrow_v[rb + CTX, pl.ds(0, 16)]
            accn1 = rrow_v[rb + CTX, pl.ds(16, 16)]
            for j in range(CTX + 1, R):
                accn0 = accn0 + rrow_v[rb + j, pl.ds(0, 16)]
                accn1 = accn1 + rrow_v[rb + j, pl.ds(16, 16)]
            c0 = crow_v[b, pl.ds(0, 16)]
            c1 = crow_v[b, pl.ds(16, 16)]
            pos_v[g * CB + b, pl.ds(0, 16)] = accp0 * c0 + accp1 * c1
            neg_v[g * CB + b, pl.ds(0, 16)] = accn0 * c0 + accn1 * c1
            return carry2

        lax.fori_loop(0, CB, elem_body, 0)

    pltpu.sync_copy(pos_v, pos_hbm.at[pl.ds(base, BPW)])
    pltpu.sync_copy(neg_v, neg_hbm.at[pl.ds(base, BPW)])


def _tc_loss_body(pos_ref, neg_ref, out_ref):
    p = jnp.sum(pos_ref[...], axis=1)
    n = jnp.sum(neg_ref[...], axis=1)
    ls = jax.nn.log_sigmoid(p) + jax.nn.log_sigmoid(-n)
    out_ref[...] = (-jnp.sum(ls) / B).reshape(1, 1)


@jax.jit
def kernel(center, context, negative, W_center, W_context):
    center = center.astype(jnp.int32)
    ctxneg = jnp.concatenate(
        [context.astype(jnp.int32), negative.astype(jnp.int32)], axis=1
    ).reshape(-1)

    mesh = plsc.VectorSubcoreMesh(core_axis_name="c", subcore_axis_name="s")

    sc_transpose = pl.kernel(
        _sc_transpose_body,
        out_type=[
            jax.ShapeDtypeStruct((VP, 128), jnp.float32),
            jax.ShapeDtypeStruct((VP, 128), jnp.float32),
        ],
        mesh=mesh,
        compiler_params=pltpu.CompilerParams(
            use_tc_tiling_on_sc=True, needs_layout_passes=False),
        scratch_types=[
            pltpu.VMEM((D, VB), jnp.float32),
            pltpu.VMEM((D, VB), jnp.float32),
            pltpu.VMEM((BR, 128), jnp.float32),
            pltpu.VMEM((BR, 128), jnp.float32),
            pltpu.VMEM((VB * 33,), jnp.float32),
            pltpu.VMEM((VTAIL // 4, 128), jnp.float32),
            pltpu.SemaphoreType.DMA,
            pltpu.SemaphoreType.DMA,
            pltpu.SemaphoreType.DMA,
            pltpu.SemaphoreType.DMA,
        ],
    )
    cen_packed, ctx_packed = sc_transpose(
        W_center.T, W_context.T,
        W_center[NBLK * VB:].reshape(VTAIL // 4, 128),
        W_context[NBLK * VB:].reshape(VTAIL // 4, 128))

    sc_expand = pl.kernel(
        _sc_expand_body,
        out_type=[
            jax.ShapeDtypeStruct((V, D), jnp.float32),
            jax.ShapeDtypeStruct((V, D), jnp.float32),
        ],
        mesh=mesh,
        compiler_params=pltpu.CompilerParams(use_tc_tiling_on_sc=False),
        scratch_types=[
            pltpu.VMEM((BR, 128), jnp.float32),
            pltpu.VMEM((BR, 128), jnp.float32),
            pltpu.VMEM((VB, D), jnp.float32),
            pltpu.VMEM((VB, D), jnp.float32),
            pltpu.SemaphoreType.DMA,
            pltpu.SemaphoreType.DMA,
            pltpu.SemaphoreType.DMA,
            pltpu.SemaphoreType.DMA,
        ],
    )
    W_center, W_context = sc_expand(cen_packed, ctx_packed)
    sc_scores = pl.kernel(
        _sc_scores_body,
        out_type=[
            jax.ShapeDtypeStruct((B, 16), jnp.float32),
            jax.ShapeDtypeStruct((B, 16), jnp.float32),
        ],
        mesh=mesh,
        compiler_params=pltpu.CompilerParams(use_tc_tiling_on_sc=False),
        scratch_types=[
            pltpu.VMEM((BPW,), jnp.int32),
            pltpu.VMEM((BPW * R,), jnp.int32),
            pltpu.VMEM((CB, D), jnp.float32),
            pltpu.VMEM((CB, D), jnp.float32),
            pltpu.VMEM((CB * R, D), jnp.float32),
            pltpu.VMEM((CB * R, D), jnp.float32),
            pltpu.VMEM((BPW, 16), jnp.float32),
            pltpu.VMEM((BPW, 16), jnp.float32),
            pltpu.SemaphoreType.DMA,
            pltpu.SemaphoreType.DMA,
            pltpu.SemaphoreType.DMA,
            pltpu.SemaphoreType.DMA,
        ],
    )
    pos, neg = sc_scores(center, ctxneg, W_center, W_context)

    loss2d = pl.pallas_call(
        _tc_loss_body,
        out_shape=jax.ShapeDtypeStruct((1, 1), jnp.float32),
    )(pos, neg)
    return loss2d[0, 0]
